# baseline (speedup 1.0000x reference)
import jax
import jax.numpy as jnp
from jax import lax
from jax.experimental import pallas as pl
from jax.experimental.pallas import tpu as pltpu
from jax.experimental.pallas import tpu_sc as plsc

N = 5000
L = 16
NPAD = 5008
NCHUNK = NPAD // L
MAXK = 128
KSTRIDE = 8
OUTPAD = 512
SCORE_THRESH = 0.05
F32 = jnp.float32
I32 = jnp.int32


def _lane():
    return lax.broadcasted_iota(I32, (L,), 0)


def _bcast(vec, lane_idx):
    return vec.at[jnp.full((L,), lane_idx, I32)].get(mode="promise_in_bounds")


def _row(r):
    return jnp.full((L,), r, I32)


def _nms_body(ph, oh, outh, pv, ov, kbox, outv, sem1, sem2):
    @pl.when((lax.axis_index("c") == 0) & (lax.axis_index("s") == 0))
    def _():
        cp1 = pltpu.async_copy(ph, pv, sem1)
        cp2 = pltpu.async_copy(oh, ov, sem2)

        zero = jnp.full((L,), 0.0, F32)
        for i in range(OUTPAD // L):
            outv[pl.ds(i * L, L)] = zero

        lane = _lane()
        cp1.wait()
        cp2.wait()

        def chunk_body(st):
            c, K, _go = st
            base = c * L
            idx = ov[pl.ds(base, L)]
            cs = plsc.load_gather(pv, [_row(4), idx])
            cx1 = plsc.load_gather(pv, [_row(0), idx])
            cy1 = plsc.load_gather(pv, [_row(1), idx])
            cx2 = plsc.load_gather(pv, [_row(2), idx])
            cy2 = plsc.load_gather(pv, [_row(3), idx])
            carea = (cx2 - cx1) * (cy2 - cy1)
            valid = cs > SCORE_THRESH

            def _iou_kill(bx1, by1, bx2, by2, barea):
                w = jnp.maximum(
                    jnp.minimum(cx2, bx2) - jnp.maximum(cx1, bx1), 0.0)
                h = jnp.maximum(
                    jnp.minimum(cy2, by2) - jnp.maximum(cy1, by1), 0.0)
                inter = w * h
                union = barea + carea - inter
                return inter > 0.5 * union

            def kept_body(k4, supp):
                v0 = kbox[pl.ds(k4 * (4 * KSTRIDE), L)]
                v1 = kbox[pl.ds(k4 * (4 * KSTRIDE) + L, L)]
                sa = _iou_kill(v0[0], v0[1], v0[2], v0[3], v0[4])
                sb = _iou_kill(v0[8], v0[9], v0[10], v0[11], v0[12])
                sc = _iou_kill(v1[0], v1[1], v1[2], v1[3], v1[4])
                sd = _iou_kill(v1[8], v1[9], v1[10], v1[11], v1[12])
                sb = sb & (4 * k4 + 1 < K)
                sc = sc & (4 * k4 + 2 < K)
                sd = sd & (4 * k4 + 3 < K)
                return supp | jnp.where(sa | sb | sc | sd, 1, 0)

            supp = lax.fori_loop(0, (K + 3) // 4, kept_body,
                                 jnp.full((L,), 0, I32))
            alive = jnp.where(valid & (supp == 0), 1, 0)

            kills = []
            for l in range(L - 1):
                k = _iou_kill(_bcast(cx1, l), _bcast(cy1, l),
                              _bcast(cx2, l), _bcast(cy2, l),
                              _bcast(carea, l))
                kills.append(k & (lane > l))
            surv = alive
            for l in range(L - 1):
                on = _bcast(surv, l) != 0
                surv = jnp.where(kills[l] & on, 0, surv)
            survm = surv != 0

            pos = (K + plsc.cumsum(surv) - 1) * KSTRIDE
            plsc.store_scatter(kbox, [pos], cx1, mask=survm)
            plsc.store_scatter(kbox, [pos + 1], cy1, mask=survm)
            plsc.store_scatter(kbox, [pos + 2], cx2, mask=survm)
            plsc.store_scatter(kbox, [pos + 3], cy2, mask=survm)
            plsc.store_scatter(kbox, [pos + 4], carea, mask=survm)
            plsc.store_scatter(kbox, [pos + 5], cs, mask=survm)
            Knew = K + plsc.all_reduce_population_count(survm)[0]

            go = (Knew < 100) & (c + 1 < NCHUNK) & (cs[L - 1] > SCORE_THRESH)
            return c + 1, Knew, go

        def chunk_cond(st):
            _c, _K, go = st
            return go

        _, kfin, _ = lax.while_loop(
            chunk_cond, chunk_body,
            (jnp.int32(0), jnp.int32(0), jnp.bool_(True)))

        for rc in range(7):
            off = rc * L
            rows = lane + off
            m = (rows < 100) & (rows < kfin)
            for col, field in enumerate((0, 1, 2, 3, 5)):
                vals = plsc.load_gather(kbox, [rows * KSTRIDE + field], mask=m)
                plsc.store_scatter(outv, [rows * 5 + col], vals, mask=m)
        pltpu.sync_copy(outv, outh)


@jax.jit
def _nms_sc(packed, order):
    mesh = plsc.VectorSubcoreMesh(core_axis_name="c", subcore_axis_name="s",
                                  num_cores=1)
    return pl.kernel(
        _nms_body,
        out_type=jax.ShapeDtypeStruct((OUTPAD,), F32),
        mesh=mesh,
        scratch_types=[pltpu.VMEM((5, NPAD), F32),
                       pltpu.VMEM((NPAD,), I32),
                       pltpu.VMEM((MAXK * KSTRIDE,), F32),
                       pltpu.VMEM((OUTPAD,), F32),
                       pltpu.SemaphoreType.DMA,
                       pltpu.SemaphoreType.DMA],
        compiler_params=pltpu.CompilerParams(needs_layout_passes=False),
    )(packed, order)


def kernel(boxes, scores):
    order = jnp.argsort(-scores).astype(jnp.int32)
    op = jnp.concatenate([order, jnp.arange(N, NPAD, dtype=jnp.int32)])
    packed = jnp.concatenate(
        [boxes.T, scores[None, :]], axis=0)
    packed = jnp.pad(packed, ((0, 0), (0, NPAD - N)))
    out = _nms_sc(packed, op)
    return out[:500].reshape(100, 5)
